# Initial kernel scaffold; baseline (speedup 1.0000x reference)
import jax
import jax.numpy as jnp
from jax.experimental import pallas as pl


def kernel(x, edge_index, edge_weight, W11, b11, W12, b12, W21, b21, W22, b22, W31, b31, W32, b32):
    raise NotImplementedError("write your pallas kernel here")


# speedup vs baseline: 5.5056x; 5.5056x over previous
import functools

import jax
import jax.numpy as jnp
from jax import lax
from jax.experimental import pallas as pl
from jax.experimental.pallas import tpu as pltpu
from jax.experimental.pallas import tpu_sc as plsc

N_NODES = 10000
N_EDGES = 320000
D = 64
NC = 2
NS = 16
NW = NC * NS
EPW = N_EDGES // NW
CH = 80
NCHUNK = EPW // CH
RPT = N_NODES // NS
ZROWS = 125

_mesh = plsc.VectorSubcoreMesh(
    core_axis_name="c", subcore_axis_name="s", num_cores=NC, num_subcores=NS)


def _seg_body(y_hbm, src_hbm, dst_hbm, w_hbm, out_hbm,
              src_v, w_v, dsti_v, rows_v, zero_v, acc, sem):
    c = lax.axis_index("c")
    s = lax.axis_index("s")
    wid = c * NS + s
    base = wid * EPW

    zvec = jnp.zeros((16,), jnp.float32)

    @pl.loop(0, ZROWS)
    def _(r):
        for dd in range(D // 16):
            zero_v[r, pl.ds(dd * 16, 16)] = zvec

    @pl.loop(0, RPT // ZROWS)
    def _(k):
        pltpu.sync_copy(zero_v, acc.at[pl.ds(s * RPT + k * ZROWS, ZROWS)])

    pltpu.sync_copy(src_hbm.at[pl.ds(base, EPW)], src_v)
    pltpu.sync_copy(w_hbm.at[pl.ds(base, EPW)], w_v)

    plsc.subcore_barrier()

    @pl.loop(0, NCHUNK)
    def _(k):
        off = k * CH
        pltpu.sync_copy(dst_hbm.at[pl.ds(base + off, CH)], dsti_v)
        pltpu.async_copy(y_hbm.at[src_v.at[pl.ds(off, CH)]], rows_v, sem).wait()

        @pl.loop(0, CH)
        def _(r):
            wb = plsc.load_gather(
                w_v, [jnp.full((16,), off + r, dtype=jnp.int32)])
            for dd in range(D // 16):
                sl = pl.ds(dd * 16, 16)
                rows_v[r, sl] = rows_v[r, sl] * wb

        pltpu.sync_copy(rows_v, acc.at[dsti_v], add=True)

    plsc.subcore_barrier()
    pltpu.sync_copy(acc.at[pl.ds(s * RPT, RPT)],
                    out_hbm.at[c, pl.ds(s * RPT, RPT)])


_seg_sum = pl.kernel(
    _seg_body,
    out_type=jax.ShapeDtypeStruct((NC, N_NODES, D), jnp.float32),
    mesh=_mesh,
    scratch_types=[
        pltpu.VMEM((EPW,), jnp.int32),
        pltpu.VMEM((EPW,), jnp.float32),
        pltpu.VMEM((CH,), jnp.int32),
        pltpu.VMEM((CH, D), jnp.float32),
        pltpu.VMEM((ZROWS, D), jnp.float32),
        pltpu.VMEM_SHARED((N_NODES, D), jnp.float32),
        pltpu.SemaphoreType.DMA,
    ],
    compiler_params=pltpu.CompilerParams(
        use_tc_tiling_on_sc=False, needs_layout_passes=False),
)


def _proj_body(x_ref, w_ref, o_ref):
    o_ref[...] = jnp.dot(x_ref[...], w_ref[...],
                         preferred_element_type=jnp.float32)


def _stage1_body(y_ref, p_ref, b1_ref, w2_ref, b2_ref, o_ref):
    u = jnp.maximum(y_ref[...] + p_ref[0] + p_ref[1] + b1_ref[...], 0.0)
    h = jnp.dot(u, w2_ref[...], preferred_element_type=jnp.float32) + b2_ref[...]
    o_ref[...] = jnp.maximum(h, 0.0)


def _stage23_body(final_relu, h_ref, p_ref, w1_ref, b1_ref, w2_ref, b2_ref, o_ref):
    z = h_ref[...] + p_ref[0] + p_ref[1]
    u = jnp.maximum(
        jnp.dot(z, w1_ref[...], preferred_element_type=jnp.float32) + b1_ref[...],
        0.0)
    h = jnp.dot(u, w2_ref[...], preferred_element_type=jnp.float32) + b2_ref[...]
    o_ref[...] = jnp.maximum(h, 0.0) if final_relu else h


def _tc_call(body, out_dim):
    return pl.pallas_call(
        body, out_shape=jax.ShapeDtypeStruct((N_NODES, out_dim), jnp.float32))


def kernel(x, edge_index, edge_weight, W11, b11, W12, b12,
           W21, b21, W22, b22, W31, b31, W32, b32):
    src = edge_index[0].astype(jnp.int32)
    dst = edge_index[1].astype(jnp.int32)
    w = edge_weight.astype(jnp.float32)

    y1 = _tc_call(_proj_body, 64)(x, W11)
    p1 = _seg_sum(y1, src, dst, w)
    h1 = _tc_call(_stage1_body, 64)(
        y1, p1, b11.reshape(1, 64), W12, b12.reshape(1, 64))

    p2 = _seg_sum(h1, src, dst, w)
    h2 = _tc_call(functools.partial(_stage23_body, True), 64)(
        h1, p2, W21, b21.reshape(1, 128), W22, b22.reshape(1, 64))

    p3 = _seg_sum(h2, src, dst, w)
    out = _tc_call(functools.partial(_stage23_body, False), 128)(
        h2, p3, W31, b31.reshape(1, 128), W32, b32.reshape(1, 128))
    return out


# speedup vs baseline: 17.2683x; 3.1365x over previous
import functools

import jax
import jax.numpy as jnp
from jax import lax
from jax.experimental import pallas as pl
from jax.experimental.pallas import tpu as pltpu
from jax.experimental.pallas import tpu_sc as plsc

N_NODES = 10000
N_EDGES = 320000
D = 64
NC = 2
NS = 16
NW = NC * NS
EPW = N_EDGES // NW
CH = 80
NCHUNK = EPW // CH
RPT = N_NODES // NS
ZROWS = 125

_mesh = plsc.VectorSubcoreMesh(
    core_axis_name="c", subcore_axis_name="s", num_cores=NC, num_subcores=NS)


NBUF = 4


def _seg_body(y_hbm, src_hbm, dst_hbm, w_hbm, out_hbm,
              src_v, w_v, dsti_v, r0, r1, r2, r3, zero_v, acc,
              g0, g1, g2, g3, s0, s1, s2, s3):
    rows = [r0, r1, r2, r3]
    gsem = [g0, g1, g2, g3]
    ssem = [s0, s1, s2, s3]
    c = lax.axis_index("c")
    s = lax.axis_index("s")
    wid = c * NS + s
    base = wid * EPW

    zvec = jnp.zeros((16,), jnp.float32)

    @pl.loop(0, ZROWS)
    def _(r):
        for dd in range(D // 16):
            zero_v[r, pl.ds(dd * 16, 16)] = zvec

    @pl.loop(0, RPT // ZROWS)
    def _(k):
        pltpu.sync_copy(zero_v, acc.at[pl.ds(s * RPT + k * ZROWS, ZROWS)])

    pltpu.sync_copy(src_hbm.at[pl.ds(base, EPW)], src_v)
    pltpu.sync_copy(w_hbm.at[pl.ds(base, EPW)], w_v)
    pltpu.sync_copy(dst_hbm.at[wid], dsti_v)

    plsc.subcore_barrier()

    def start_gather(k, b):
        pltpu.async_copy(y_hbm.at[src_v.at[pl.ds(k * CH, CH)]], rows[b],
                         gsem[b])

    def wait_gather(k, b):
        pltpu.make_async_copy(y_hbm.at[src_v.at[pl.ds(k * CH, CH)]], rows[b],
                              gsem[b]).wait()

    def start_scatter(k, b):
        pltpu.async_copy(rows[b], acc.at[dsti_v.at[k]], ssem[b], add=True)

    def wait_scatter(k, b):
        pltpu.make_async_copy(rows[b], acc.at[dsti_v.at[k]], ssem[b]).wait()

    def multiply(k, b):
        buf = rows[b]

        @plsc.parallel_loop(0, CH, unroll=8)
        def _(r):
            wb = plsc.load_gather(
                w_v, [jnp.full((16,), k * CH + r, dtype=jnp.int32)])
            for dd in range(D // 16):
                sl = pl.ds(dd * 16, 16)
                buf[r, sl] = buf[r, sl] * wb

    for b in range(NBUF - 1):
        start_gather(b, b)

    ngrp = (NCHUNK + NBUF - 1) // NBUF

    @pl.loop(0, ngrp)
    def _(j):
        for i in range(NBUF):
            k = j * NBUF + i

            @pl.when(k < NCHUNK)
            def _():
                wait_gather(k, i)
                multiply(k, i)
                start_scatter(k, i)

            @pl.when(jnp.logical_and(k >= 1, k <= NCHUNK - 1))
            def _():
                wait_scatter(k - 1, (i - 1) % NBUF)

            @pl.when(k + NBUF - 1 < NCHUNK)
            def _():
                start_gather(k + NBUF - 1, (i + NBUF - 1) % NBUF)

    wait_scatter(NCHUNK - 1, (NCHUNK - 1) % NBUF)

    plsc.subcore_barrier()
    pltpu.sync_copy(acc.at[pl.ds(s * RPT, RPT)],
                    out_hbm.at[c, pl.ds(s * RPT, RPT)])


_seg_sum = pl.kernel(
    _seg_body,
    out_type=jax.ShapeDtypeStruct((NC, N_NODES, D), jnp.float32),
    mesh=_mesh,
    scratch_types=[
        pltpu.VMEM((EPW,), jnp.int32),
        pltpu.VMEM((EPW,), jnp.float32),
        pltpu.VMEM((NCHUNK, CH), jnp.int32),
        pltpu.VMEM((CH, D), jnp.float32),
        pltpu.VMEM((CH, D), jnp.float32),
        pltpu.VMEM((CH, D), jnp.float32),
        pltpu.VMEM((CH, D), jnp.float32),
        pltpu.VMEM((ZROWS, D), jnp.float32),
        pltpu.VMEM_SHARED((N_NODES, D), jnp.float32),
    ] + [pltpu.SemaphoreType.DMA] * (2 * NBUF),
    compiler_params=pltpu.CompilerParams(
        use_tc_tiling_on_sc=False, needs_layout_passes=False),
)


def _proj_body(x_ref, w_ref, o_ref):
    o_ref[...] = jnp.dot(x_ref[...], w_ref[...],
                         preferred_element_type=jnp.float32)


def _stage1_body(y_ref, p_ref, b1_ref, w2_ref, b2_ref, o_ref):
    u = jnp.maximum(y_ref[...] + p_ref[0] + p_ref[1] + b1_ref[...], 0.0)
    h = jnp.dot(u, w2_ref[...], preferred_element_type=jnp.float32) + b2_ref[...]
    o_ref[...] = jnp.maximum(h, 0.0)


def _stage23_body(final_relu, h_ref, p_ref, w1_ref, b1_ref, w2_ref, b2_ref, o_ref):
    z = h_ref[...] + p_ref[0] + p_ref[1]
    u = jnp.maximum(
        jnp.dot(z, w1_ref[...], preferred_element_type=jnp.float32) + b1_ref[...],
        0.0)
    h = jnp.dot(u, w2_ref[...], preferred_element_type=jnp.float32) + b2_ref[...]
    o_ref[...] = jnp.maximum(h, 0.0) if final_relu else h


def _tc_call(body, out_dim):
    return pl.pallas_call(
        body, out_shape=jax.ShapeDtypeStruct((N_NODES, out_dim), jnp.float32))


def kernel(x, edge_index, edge_weight, W11, b11, W12, b12,
           W21, b21, W22, b22, W31, b31, W32, b32):
    src = edge_index[0].astype(jnp.int32)
    dst = edge_index[1].astype(jnp.int32).reshape(NW, NCHUNK, CH)
    w = edge_weight.astype(jnp.float32)

    y1 = _tc_call(_proj_body, 64)(x, W11)
    p1 = _seg_sum(y1, src, dst, w)
    h1 = _tc_call(_stage1_body, 64)(
        y1, p1, b11.reshape(1, 64), W12, b12.reshape(1, 64))

    p2 = _seg_sum(h1, src, dst, w)
    h2 = _tc_call(functools.partial(_stage23_body, True), 64)(
        h1, p2, W21, b21.reshape(1, 128), W22, b22.reshape(1, 64))

    p3 = _seg_sum(h2, src, dst, w)
    out = _tc_call(functools.partial(_stage23_body, False), 128)(
        h2, p3, W31, b31.reshape(1, 128), W32, b32.reshape(1, 128))
    return out
